# baseline (speedup 1.0000x reference)
import jax
import jax.numpy as jnp
from jax.experimental import pallas as pl
from jax.experimental.pallas import tpu as pltpu

N = 4096
FEATS = 128
HID = 64
LABEL = 10
BLK = 512
NSTEP = N // (2 * BLK)


def _gcn_kernel(v_ref, adja_ref, adjb_ref, w0_ref, wouta_ref, woutb_ref,
                bout_ref, out_ref, support_ref):
    i = pl.program_id(0)

    @pl.when(i == 0)
    def _init():
        support_ref[:] = jnp.dot(v_ref[:], w0_ref[:],
                                 preferred_element_type=jnp.float32)
        out_ref[:] = bout_ref[:]

    sup = support_ref[:]
    ha = jnp.dot(adja_ref[:], sup, preferred_element_type=jnp.float32)
    hb = jnp.dot(adjb_ref[:], sup, preferred_element_type=jnp.float32)
    sa = jnp.sum(jnp.maximum(ha, 0.0), axis=1)[None, :]
    sb = jnp.sum(jnp.maximum(hb, 0.0), axis=1)[None, :]
    dn = (((1,), (1,)), ((), ()))
    out_ref[:] += (
        jax.lax.dot_general(sa, wouta_ref[:], dn,
                            preferred_element_type=jnp.float32)
        + jax.lax.dot_general(sb, woutb_ref[:], dn,
                              preferred_element_type=jnp.float32))


def kernel(v, adj, W0, W_out, b_out):
    out = pl.pallas_call(
        _gcn_kernel,
        grid=(NSTEP,),
        in_specs=[
            pl.BlockSpec((N, FEATS), lambda i: (0, 0)),
            pl.BlockSpec((BLK, N), lambda i: (i, 0)),
            pl.BlockSpec((BLK, N), lambda i: (i + NSTEP, 0)),
            pl.BlockSpec((FEATS, HID), lambda i: (0, 0)),
            pl.BlockSpec((LABEL, BLK), lambda i: (0, i)),
            pl.BlockSpec((LABEL, BLK), lambda i: (0, i + NSTEP)),
            pl.BlockSpec((1, LABEL), lambda i: (0, 0)),
        ],
        out_specs=pl.BlockSpec((1, LABEL), lambda i: (0, 0)),
        out_shape=jax.ShapeDtypeStruct((1, LABEL), jnp.float32),
        scratch_shapes=[pltpu.VMEM((N, HID), jnp.float32)],
    )(v, adj, adj, W0, W_out, W_out, b_out.reshape(1, LABEL))
    return out.reshape(LABEL)


# speedup vs baseline: 1.1418x; 1.1418x over previous
import jax
import jax.numpy as jnp
from jax.experimental import pallas as pl
from jax.experimental.pallas import tpu as pltpu

N = 4096
FEATS = 128
HID = 64
LABEL = 10
BLK = 512


def _gcn_kernel(v_ref, adj_ref, w0_ref, wout_ref, bout_ref, out_ref,
                support_ref):
    i = pl.program_id(0)

    @pl.when(i == 0)
    def _init():
        support_ref[:] = jnp.dot(v_ref[:], w0_ref[:],
                                 preferred_element_type=jnp.float32)
        out_ref[:] = bout_ref[:]

    ht = jax.lax.dot_general(
        support_ref[:], adj_ref[:], (((0,), (1,)), ((), ())),
        preferred_element_type=jnp.float32)
    s = jnp.sum(jnp.maximum(ht, 0.0), axis=0)[None, :]
    out_ref[:] += jax.lax.dot_general(
        s, wout_ref[:], (((1,), (1,)), ((), ())),
        preferred_element_type=jnp.float32)


def kernel(v, adj, W0, W_out, b_out):
    out = pl.pallas_call(
        _gcn_kernel,
        grid=(N // BLK,),
        in_specs=[
            pl.BlockSpec((N, FEATS), lambda i: (0, 0)),
            pl.BlockSpec((BLK, N), lambda i: (i, 0)),
            pl.BlockSpec((FEATS, HID), lambda i: (0, 0)),
            pl.BlockSpec((LABEL, BLK), lambda i: (0, i)),
            pl.BlockSpec((1, LABEL), lambda i: (0, 0)),
        ],
        out_specs=pl.BlockSpec((1, LABEL), lambda i: (0, 0)),
        out_shape=jax.ShapeDtypeStruct((1, LABEL), jnp.float32),
        scratch_shapes=[pltpu.VMEM((N, HID), jnp.float32)],
    )(v, adj, W0, W_out, b_out.reshape(1, LABEL))
    return out.reshape(LABEL)
